# baseline (speedup 1.0000x reference)
import functools

import jax
import jax.numpy as jnp
from jax import lax
from jax.experimental import pallas as pl
from jax.experimental.pallas import tpu as pltpu
from jax.experimental.pallas import tpu_sc as plsc

N_TASKS = 1024
N_SPLITS = 8
N_SKILLS = 64
D = N_SPLITS * N_SKILLS
DP = D // 2
B = 16384
EPS = 1e-12

_NC = 2
_NS = 16
_NW = _NC * _NS

_B_PER_W = B // _NW
_CH = 64
_NCH = _B_PER_W // _CH
_NBUF = 2


def _norm_body(table_ref, out_ref):
    x = table_ref[...]
    s = 1.0 / (1.0 + jnp.exp(-x))
    for g in range(N_SPLITS):
        sl = slice(g * N_SKILLS, (g + 1) * N_SKILLS)
        grp = s[:, sl]
        tot = jnp.sum(grp, axis=1, keepdims=True) + EPS
        out_ref[:, sl] = (grp * (1.0 / tot)).astype(jnp.bfloat16)


_normalize = pl.pallas_call(
    _norm_body,
    out_shape=jax.ShapeDtypeStruct((N_TASKS, D), jnp.bfloat16),
)


def _gather_body(tab_hbm, ids_hbm, out_hbm, idx_v,
                 rb0, rb1, g0, g1, s0, s1):
    rbufs = [rb0, rb1]
    gsems = [g0, g1]
    ssems = [s0, s1]

    wid = lax.axis_index("s") * _NC + lax.axis_index("c")
    base = wid * _B_PER_W
    pltpu.sync_copy(ids_hbm.at[wid], idx_v)

    def fire_gather(ch):
        b = ch % _NBUF
        return pltpu.async_copy(tab_hbm.at[idx_v.at[ch]], rbufs[b], gsems[b])

    gops = [None] * _NCH
    sops = [None] * _NCH
    for ch in range(_NBUF):
        gops[ch] = fire_gather(ch)
    for ch in range(_NCH):
        b = ch % _NBUF
        gops[ch].wait()
        if ch >= 1:
            sops[ch - 1].wait()
            nxt = ch - 1 + _NBUF
            if nxt < _NCH:
                gops[nxt] = fire_gather(nxt)
        sops[ch] = pltpu.async_copy(
            rbufs[b], out_hbm.at[pl.ds(base + ch * _CH, _CH)], ssems[b])
    sops[_NCH - 1].wait()


_mesh = plsc.VectorSubcoreMesh(core_axis_name="c", subcore_axis_name="s")

_gather = functools.partial(
    pl.kernel,
    mesh=_mesh,
    out_type=jax.ShapeDtypeStruct((B, DP), jnp.float32),
    scratch_types=[
        pltpu.VMEM((_NCH, _CH), jnp.int32),
        pltpu.VMEM((_CH, DP), jnp.float32),
        pltpu.VMEM((_CH, DP), jnp.float32),
        pltpu.SemaphoreType.DMA,
        pltpu.SemaphoreType.DMA,
        pltpu.SemaphoreType.DMA,
        pltpu.SemaphoreType.DMA,
    ],
)(_gather_body)


@jax.jit
def kernel(module_logits, task_ids):
    w16 = _normalize(module_logits)
    packed = lax.bitcast_convert_type(
        w16.reshape(N_TASKS, DP, 2), jnp.float32)
    ids = task_ids.astype(jnp.int32).reshape(_NW, _NCH, _CH)
    out = _gather(packed, ids)
    u16 = lax.bitcast_convert_type(out, jnp.bfloat16)
    return u16.reshape(B, N_SPLITS, N_SKILLS).astype(jnp.float32)


# speedup vs baseline: 1.3879x; 1.3879x over previous
import functools

import jax
import jax.numpy as jnp
from jax import lax
from jax.experimental import pallas as pl
from jax.experimental.pallas import tpu as pltpu
from jax.experimental.pallas import tpu_sc as plsc

N_TASKS = 1024
N_SPLITS = 8
N_SKILLS = 64
D = N_SPLITS * N_SKILLS
DP = D // 2
B = 16384
EPS = 1e-12

_NC = 2
_NS = 16
_NW = _NC * _NS

_B_PER_W = B // _NW
_CH = 64
_NCH = _B_PER_W // _CH
_NBUF = 2


def _norm_body(table_ref, out_ref):
    x = table_ref[...]
    s = 1.0 / (1.0 + jnp.exp(-x))
    w = []
    for g in range(N_SPLITS):
        sl = slice(g * N_SKILLS, (g + 1) * N_SKILLS)
        grp = s[:, sl]
        tot = jnp.sum(grp, axis=1, keepdims=True) + EPS
        w.append((grp * (1.0 / tot)).astype(jnp.bfloat16).astype(jnp.float32))
    for g in range(N_SPLITS // 2):
        hi = lax.bitcast_convert_type(w[g], jnp.uint32)
        lo = lax.bitcast_convert_type(w[g + N_SPLITS // 2], jnp.uint32)
        packed = hi | (lo >> 16)
        out_ref[:, g * N_SKILLS:(g + 1) * N_SKILLS] = (
            lax.bitcast_convert_type(packed, jnp.float32))


_normalize = pl.pallas_call(
    _norm_body,
    out_shape=jax.ShapeDtypeStruct((N_TASKS, DP), jnp.float32),
)


def _gather_body(tab_hbm, ids_hbm, out_hbm, idx_v,
                 rb0, rb1, g0, g1, s0, s1):
    rbufs = [rb0, rb1]
    gsems = [g0, g1]
    ssems = [s0, s1]

    wid = lax.axis_index("s") * _NC + lax.axis_index("c")
    base = wid * _B_PER_W
    pltpu.sync_copy(ids_hbm.at[wid], idx_v)

    def fire_gather(ch):
        b = ch % _NBUF
        return pltpu.async_copy(tab_hbm.at[idx_v.at[ch]], rbufs[b], gsems[b])

    gops = [None] * _NCH
    sops = [None] * _NCH
    for ch in range(_NBUF):
        gops[ch] = fire_gather(ch)
    for ch in range(_NCH):
        b = ch % _NBUF
        gops[ch].wait()
        if ch >= 1:
            sops[ch - 1].wait()
            nxt = ch - 1 + _NBUF
            if nxt < _NCH:
                gops[nxt] = fire_gather(nxt)
        sops[ch] = pltpu.async_copy(
            rbufs[b], out_hbm.at[pl.ds(base + ch * _CH, _CH)], ssems[b])
    sops[_NCH - 1].wait()


_mesh = plsc.VectorSubcoreMesh(core_axis_name="c", subcore_axis_name="s")

_gather = functools.partial(
    pl.kernel,
    mesh=_mesh,
    out_type=jax.ShapeDtypeStruct((B, DP), jnp.float32),
    scratch_types=[
        pltpu.VMEM((_NCH, _CH), jnp.int32),
        pltpu.VMEM((_CH, DP), jnp.float32),
        pltpu.VMEM((_CH, DP), jnp.float32),
        pltpu.SemaphoreType.DMA,
        pltpu.SemaphoreType.DMA,
        pltpu.SemaphoreType.DMA,
        pltpu.SemaphoreType.DMA,
    ],
)(_gather_body)


@jax.jit
def kernel(module_logits, task_ids):
    packed = _normalize(module_logits)
    ids = task_ids.astype(jnp.int32).reshape(_NW, _NCH, _CH)
    out = _gather(packed, ids)
    u = lax.bitcast_convert_type(out, jnp.uint32)
    hi = lax.bitcast_convert_type(u & jnp.uint32(0xFFFF0000), jnp.float32)
    lo = lax.bitcast_convert_type(u << 16, jnp.float32)
    return jnp.concatenate(
        [hi.reshape(B, N_SPLITS // 2, N_SKILLS),
         lo.reshape(B, N_SPLITS // 2, N_SKILLS)], axis=1)


# speedup vs baseline: 1.8710x; 1.3481x over previous
import functools

import jax
import jax.numpy as jnp
from jax import lax
from jax.experimental import pallas as pl
from jax.experimental.pallas import tpu as pltpu
from jax.experimental.pallas import tpu_sc as plsc

N_TASKS = 1024
N_SPLITS = 8
N_SKILLS = 64
D = N_SPLITS * N_SKILLS
B = 16384
EPS = 1e-12

_NC = 2
_NS = 16
_NW = _NC * _NS

_B_PER_W = B // _NW
_CH = 64
_NCH = _B_PER_W // _CH
_NBUF = 2


def _norm_body(table_ref, out_ref):
    x = table_ref[...]
    s = 1.0 / (1.0 + jnp.exp(-x))
    for g in range(N_SPLITS):
        sl = slice(g * N_SKILLS, (g + 1) * N_SKILLS)
        grp = s[:, sl]
        tot = jnp.sum(grp, axis=1, keepdims=True) + EPS
        out_ref[:, sl] = grp * (1.0 / tot)


_normalize = pl.pallas_call(
    _norm_body,
    out_shape=jax.ShapeDtypeStruct((N_TASKS, D), jnp.float32),
)


def _gather_body(norm_hbm, ids_hbm, out_hbm, idx_v,
                 rb0, rb1, g0, g1, s0, s1):
    rbufs = [rb0, rb1]
    gsems = [g0, g1]
    ssems = [s0, s1]

    wid = lax.axis_index("s") * _NC + lax.axis_index("c")
    base = wid * _B_PER_W
    pltpu.sync_copy(ids_hbm.at[wid], idx_v)

    def fire_gather(ch):
        b = ch % _NBUF
        return pltpu.async_copy(norm_hbm.at[idx_v.at[ch]], rbufs[b], gsems[b])

    gops = [None] * _NCH
    sops = [None] * _NCH
    for ch in range(_NBUF):
        gops[ch] = fire_gather(ch)
    for ch in range(_NCH):
        b = ch % _NBUF
        gops[ch].wait()
        if ch >= 1:
            sops[ch - 1].wait()
            nxt = ch - 1 + _NBUF
            if nxt < _NCH:
                gops[nxt] = fire_gather(nxt)
        sops[ch] = pltpu.async_copy(
            rbufs[b], out_hbm.at[pl.ds(base + ch * _CH, _CH)], ssems[b])
    sops[_NCH - 1].wait()


_mesh = plsc.VectorSubcoreMesh(core_axis_name="c", subcore_axis_name="s")

_gather = functools.partial(
    pl.kernel,
    mesh=_mesh,
    out_type=jax.ShapeDtypeStruct((B, D), jnp.float32),
    scratch_types=[
        pltpu.VMEM((_NCH, _CH), jnp.int32),
        pltpu.VMEM((_CH, D), jnp.float32),
        pltpu.VMEM((_CH, D), jnp.float32),
        pltpu.SemaphoreType.DMA,
        pltpu.SemaphoreType.DMA,
        pltpu.SemaphoreType.DMA,
        pltpu.SemaphoreType.DMA,
    ],
)(_gather_body)


@jax.jit
def kernel(module_logits, task_ids):
    norm = _normalize(module_logits)
    ids = task_ids.astype(jnp.int32).reshape(_NW, _NCH, _CH)
    return _gather(norm, ids).reshape(B, N_SPLITS, N_SKILLS)
